# baseline (speedup 1.0000x reference)
import functools

import jax
import jax.numpy as jnp
from jax import lax
from jax.experimental import pallas as pl
from jax.experimental.pallas import tpu as pltpu
from jax.experimental.pallas import tpu_sc as plsc

_B = 16384
_D = 64
_L = 16

_info = plsc.get_sparse_core_info()
_NC, _NS = _info.num_cores, _info.num_subcores
_NW = _NC * _NS
_N = _B // _NW
_NIDX = _N // 128


def _transe_body(h_idx_hbm, r_idx_hbm, o_idx_hbm, ent_hbm, rel_hbm, out_hbm,
                 h_idx_v, r_idx_v, o_idx_v, h_rows, r_rows, o_rows, out_v,
                 sem):
    wid = lax.axis_index("s") * _NC + lax.axis_index("c")
    ibase = wid * _NIDX

    pltpu.sync_copy(h_idx_hbm.at[pl.ds(ibase, _NIDX)], h_idx_v)
    pltpu.sync_copy(r_idx_hbm.at[pl.ds(ibase, _NIDX)], r_idx_v)
    pltpu.sync_copy(o_idx_hbm.at[pl.ds(ibase, _NIDX)], o_idx_v)

    pltpu.sync_copy(ent_hbm.at[pl.ds(0, _N)], h_rows)
    pltpu.sync_copy(rel_hbm.at[pl.ds(0, _N)], r_rows)
    pltpu.sync_copy(ent_hbm.at[pl.ds(_N, _N)], o_rows)

    iota = lax.iota(jnp.int32, _L)
    shuf = [jnp.bitwise_xor(iota, k) for k in (1, 2, 4, 8)]

    def group(g, _):
        i = g * _L
        outvec = jnp.abs(h_rows[i, pl.ds(0, _L)] + r_rows[i, pl.ds(0, _L)]
                         - o_rows[i, pl.ds(0, _L)])
        out_v[pl.ds(g * _L, _L)] = outvec
        return 0

    lax.fori_loop(0, _N // _L, group, 0)

    pltpu.sync_copy(out_v, out_hbm.at[pl.ds(wid * _N, _N)])


@jax.jit
def _transe_sc(h_idx, r_idx, o_idx, ent, rel):
    mesh = plsc.VectorSubcoreMesh(core_axis_name="c", subcore_axis_name="s")
    run = functools.partial(
        pl.kernel,
        mesh=mesh,
        compiler_params=pltpu.CompilerParams(use_tc_tiling_on_sc=False),
        out_type=jax.ShapeDtypeStruct((_B,), jnp.float32),
        scratch_types=[
            pltpu.VMEM((_NIDX, 128), jnp.int32),
            pltpu.VMEM((_NIDX, 128), jnp.int32),
            pltpu.VMEM((_NIDX, 128), jnp.int32),
            pltpu.VMEM((_N, _D), jnp.float32),
            pltpu.VMEM((_N, _D), jnp.float32),
            pltpu.VMEM((_N, _D), jnp.float32),
            pltpu.VMEM((_N,), jnp.float32),
            pltpu.SemaphoreType.DMA,
        ],
    )(_transe_body)
    return run(h_idx, r_idx, o_idx, ent, rel)


def kernel(triplets, entity_embeddings, relation_embeddings):
    h_idx = triplets[:, 0].reshape(_B // 128, 128)
    r_idx = triplets[:, 1].reshape(_B // 128, 128)
    o_idx = triplets[:, 2].reshape(_B // 128, 128)
    return _transe_sc(h_idx, r_idx, o_idx,
                      entity_embeddings, relation_embeddings)


# speedup vs baseline: 1.0017x; 1.0017x over previous
import functools

import jax
import jax.numpy as jnp
from jax import lax
from jax.experimental import pallas as pl
from jax.experimental.pallas import tpu as pltpu
from jax.experimental.pallas import tpu_sc as plsc

_B = 16384
_D = 64
_L = 16

_info = plsc.get_sparse_core_info()
_NC, _NS = _info.num_cores, _info.num_subcores
_NW = _NC * _NS
_N = _B // _NW
_NIDX = _N // 128


def _transe_body(h_idx_hbm, r_idx_hbm, o_idx_hbm, ent_hbm, rel_hbm, out_hbm,
                 h_idx_v, r_idx_v, o_idx_v, h_rows, r_rows, o_rows, out_v,
                 sem):
    wid = lax.axis_index("s") * _NC + lax.axis_index("c")
    ibase = wid * _NIDX

    pltpu.sync_copy(h_idx_hbm.at[pl.ds(ibase, _NIDX)], h_idx_v)
    pltpu.sync_copy(r_idx_hbm.at[pl.ds(ibase, _NIDX)], r_idx_v)
    pltpu.sync_copy(o_idx_hbm.at[pl.ds(ibase, _NIDX)], o_idx_v)

    copies = []
    for j in range(_NIDX):
        dst = pl.ds(j * 128, 128)
        copies.append(pltpu.async_copy(ent_hbm.at[h_idx_v.at[j]],
                                       h_rows.at[dst], sem))
        copies.append(pltpu.async_copy(rel_hbm.at[r_idx_v.at[j]],
                                       r_rows.at[dst], sem))
        copies.append(pltpu.async_copy(ent_hbm.at[o_idx_v.at[j]],
                                       o_rows.at[dst], sem))
    for c in copies:
        c.wait()

    iota = lax.iota(jnp.int32, _L)
    shuf = [jnp.bitwise_xor(iota, k) for k in (1, 2, 4, 8)]

    def group(g, _):
        outvec = jnp.zeros((_L,), jnp.float32)
        for j in range(_L):
            i = g * _L + j
            v = jnp.zeros((_L,), jnp.float32)
            for c in range(_D // _L):
                sl = pl.ds(c * _L, _L)
                v = v + jnp.abs(h_rows[i, sl] + r_rows[i, sl] - o_rows[i, sl])
            for s in shuf:
                v = v + v.at[s].get(mode="promise_in_bounds")
            outvec = jnp.where(iota == j, v, outvec)
        out_v[pl.ds(g * _L, _L)] = outvec
        return 0

    lax.fori_loop(0, _N // _L, group, 0)

    pltpu.sync_copy(out_v, out_hbm.at[pl.ds(wid * _N, _N)])


@jax.jit
def _transe_sc(h_idx, r_idx, o_idx, ent, rel):
    mesh = plsc.VectorSubcoreMesh(core_axis_name="c", subcore_axis_name="s")
    run = functools.partial(
        pl.kernel,
        mesh=mesh,
        compiler_params=pltpu.CompilerParams(use_tc_tiling_on_sc=False),
        out_type=jax.ShapeDtypeStruct((_B,), jnp.float32),
        scratch_types=[
            pltpu.VMEM((_NIDX, 128), jnp.int32),
            pltpu.VMEM((_NIDX, 128), jnp.int32),
            pltpu.VMEM((_NIDX, 128), jnp.int32),
            pltpu.VMEM((_N, _D), jnp.float32),
            pltpu.VMEM((_N, _D), jnp.float32),
            pltpu.VMEM((_N, _D), jnp.float32),
            pltpu.VMEM((_N,), jnp.float32),
            pltpu.SemaphoreType.DMA,
        ],
    )(_transe_body)
    return run(h_idx, r_idx, o_idx, ent, rel)


def kernel(triplets, entity_embeddings, relation_embeddings):
    h_idx = triplets[:, 0].reshape(_B // 128, 128)
    r_idx = triplets[:, 1].reshape(_B // 128, 128)
    o_idx = triplets[:, 2].reshape(_B // 128, 128)
    return _transe_sc(h_idx, r_idx, o_idx,
                      entity_embeddings, relation_embeddings)
